# Initial kernel scaffold; baseline (speedup 1.0000x reference)
import jax
import jax.numpy as jnp
from jax.experimental import pallas as pl


def kernel(users, items, adj_entity, adj_relation, user_emb, entity_emb, relation_emb, W0, b0, W1, b1):
    raise NotImplementedError("write your pallas kernel here")


# speedup vs baseline: 7.2116x; 7.2116x over previous
import functools

import jax
import jax.numpy as jnp
from jax import lax
from jax.experimental import pallas as pl
from jax.experimental.pallas import tpu as pltpu
from jax.experimental.pallas import tpu_sc as plsc

B = 4096
D = 32
K = 16
NR_PAD = 128

NC = 2
NS = 16
NW = NC * NS
NB = B // NW

_f32 = jnp.float32
_i32 = jnp.int32


def _mesh():
    return plsc.VectorSubcoreMesh(
        core_axis_name="c", subcore_axis_name="s", num_cores=NC, num_subcores=NS)


_SC_PARAMS = pltpu.CompilerParams(use_tc_tiling_on_sc=False,
                                  needs_layout_passes=False)


def _wid():
    return lax.axis_index("s") * NC + lax.axis_index("c")


def _k1_body(users_hbm, items_hbm, uemb_hbm, eemb_hbm, u_out, ev0_out,
             uidx_v, iidx_v, u_v, e_v, sem1, sem2):
    base = _wid() * NB
    pltpu.sync_copy(users_hbm.at[pl.ds(base, NB)], uidx_v)
    pltpu.sync_copy(items_hbm.at[pl.ds(base, NB)], iidx_v)
    c1 = pltpu.async_copy(uemb_hbm.at[uidx_v], u_v, sem1)
    c2 = pltpu.async_copy(eemb_hbm.at[iidx_v], e_v, sem2)
    c1.wait()
    c2.wait()
    pltpu.sync_copy(u_v, u_out.at[pl.ds(base, NB)])
    pltpu.sync_copy(e_v, ev0_out.at[pl.ds(base, NB)])


def _k1(users, items, user_emb, entity_emb):
    fn = pl.kernel(
        _k1_body,
        out_type=(jax.ShapeDtypeStruct((B, D), _f32),
                  jax.ShapeDtypeStruct((B, D), _f32)),
        mesh=_mesh(),
        compiler_params=_SC_PARAMS,
        scratch_types=[
            pltpu.VMEM((NB,), _i32),
            pltpu.VMEM((NB,), _i32),
            pltpu.VMEM((NB, D), _f32),
            pltpu.VMEM((NB, D), _f32),
            pltpu.SemaphoreType.DMA,
            pltpu.SemaphoreType.DMA,
        ],
    )
    return fn(users, items, user_emb, entity_emb)


def _k2_body(u_ref, r_ref, o_ref):
    o_ref[...] = lax.dot_general(
        u_ref[...], r_ref[...], (((1,), (1,)), ((), ())),
        preferred_element_type=_f32) * (1.0 / D)


def _k2(u, rel_pad):
    return pl.pallas_call(
        _k2_body,
        out_shape=jax.ShapeDtypeStruct((B, NR_PAD), _f32),
    )(u, rel_pad)


def _k3a_body(p_hbm, items_hbm, adje_hbm, adjr_hbm, eemb_hbm,
              ev1_out, e2_out, s1_out, s0_out,
              it_v, e1_v, r0_v, p_v, s0_v, e2_b, r1_b, ev1_b, s1_b,
              sem_ids, sem_b):
    base = _wid() * NB
    pltpu.sync_copy(items_hbm.at[pl.ds(base, NB)], it_v)
    c1 = pltpu.async_copy(adje_hbm.at[it_v], e1_v, sem_ids)
    c2 = pltpu.async_copy(adjr_hbm.at[it_v], r0_v, sem_ids)
    pltpu.sync_copy(p_hbm.at[pl.ds(base, NB)], p_v)
    c1.wait()
    c2.wait()

    def body(b, carry):
        gb = base + b
        idx16 = e1_v[b]
        d1 = pltpu.async_copy(adje_hbm.at[idx16], e2_b, sem_b)
        d2 = pltpu.async_copy(adjr_hbm.at[idx16], r1_b, sem_b)
        d3 = pltpu.async_copy(eemb_hbm.at[idx16], ev1_b, sem_b)
        d1.wait()
        d2.wait()
        d3.wait()
        bvec = jnp.full((K,), b, _i32)
        for j in range(K):
            s1_b[j] = plsc.load_gather(p_v, [bvec, r1_b[j]])
        s0_v[b] = plsc.load_gather(p_v, [bvec, r0_v[b]])
        pltpu.sync_copy(e2_b, e2_out.at[gb])
        pltpu.sync_copy(ev1_b, ev1_out.at[gb])
        pltpu.sync_copy(s1_b, s1_out.at[gb])
        return carry

    lax.fori_loop(0, NB, body, 0)
    pltpu.sync_copy(s0_v, s0_out.at[pl.ds(base, NB)])


def _k3a(p, items, adj_entity, adj_relation, entity_emb):
    fn = pl.kernel(
        _k3a_body,
        out_type=(jax.ShapeDtypeStruct((B, K, D), _f32),
                  jax.ShapeDtypeStruct((B, K, K), _i32),
                  jax.ShapeDtypeStruct((B, K, K), _f32),
                  jax.ShapeDtypeStruct((B, K), _f32)),
        mesh=_mesh(),
        compiler_params=_SC_PARAMS,
        scratch_types=[
            pltpu.VMEM((NB,), _i32),
            pltpu.VMEM((NB, K), _i32),
            pltpu.VMEM((NB, K), _i32),
            pltpu.VMEM((NB, NR_PAD), _f32),
            pltpu.VMEM((NB, K), _f32),
            pltpu.VMEM((K, K), _i32),
            pltpu.VMEM((K, K), _i32),
            pltpu.VMEM((K, D), _f32),
            pltpu.VMEM((K, K), _f32),
            pltpu.SemaphoreType.DMA,
            pltpu.SemaphoreType.DMA,
        ],
    )
    return fn(p, items, adj_entity, adj_relation, entity_emb)


_RPT = B * K * K // NW
_CH = 128


def _k3b_body(idx_hbm, eemb_hbm, out_hbm, idx_v, rows_v, sem):
    rbase = _wid() * _RPT

    def body(i, carry):
        off = rbase + i * _CH
        pltpu.sync_copy(idx_hbm.at[pl.ds(off, _CH)], idx_v)
        pltpu.async_copy(eemb_hbm.at[idx_v], rows_v, sem).wait()
        pltpu.sync_copy(rows_v, out_hbm.at[pl.ds(off, _CH)])
        return carry

    lax.fori_loop(0, _RPT // _CH, body, 0)


def _k3b(e2_flat, entity_emb):
    fn = pl.kernel(
        _k3b_body,
        out_type=jax.ShapeDtypeStruct((B * K * K, D), _f32),
        mesh=_mesh(),
        compiler_params=_SC_PARAMS,
        scratch_types=[
            pltpu.VMEM((_CH,), _i32),
            pltpu.VMEM((_CH, D), _f32),
            pltpu.SemaphoreType.DMA,
        ],
    )
    return fn(e2_flat, entity_emb)


_BS = 128


def _k4_body(ev0_ref, ev1_ref, ev2_ref, s0_ref, s1_ref,
             w0_ref, c0_ref, w1_ref, c1_ref, out_ref):
    ev0 = ev0_ref[...]
    ev1 = ev1_ref[...]
    ev2 = ev2_ref[...]
    W0 = w0_ref[...]
    b0 = c0_ref[...]
    W1 = w1_ref[...]
    b1 = c1_ref[...]

    wt1 = jax.nn.softmax(s1_ref[...], axis=-1)
    agg1 = jnp.sum(wt1[..., None] * ev2, axis=2)
    x1 = (ev1 + agg1).reshape(_BS * K, D)
    h1 = jax.nn.relu(
        lax.dot_general(x1, W0, (((1,), (1,)), ((), ())),
                        preferred_element_type=_f32) + b0).reshape(_BS, K, D)

    wt0 = jax.nn.softmax(s0_ref[...], axis=-1)
    agg0 = jnp.sum(wt0[..., None] * ev1, axis=1)
    h0 = jax.nn.relu(
        lax.dot_general(ev0 + agg0, W0, (((1,), (1,)), ((), ())),
                        preferred_element_type=_f32) + b0)

    aggf = jnp.sum(wt0[..., None] * h1, axis=1)
    out_ref[...] = jnp.tanh(
        lax.dot_general(h0 + aggf, W1, (((1,), (1,)), ((), ())),
                        preferred_element_type=_f32) + b1)


def _k4(ev0, ev1, ev2, s0, s1, W0, b0, W1, b1):
    nblk = B // _BS
    return pl.pallas_call(
        _k4_body,
        grid=(nblk,),
        in_specs=[
            pl.BlockSpec((_BS, D), lambda i: (i, 0)),
            pl.BlockSpec((_BS, K, D), lambda i: (i, 0, 0)),
            pl.BlockSpec((_BS, K, K, D), lambda i: (i, 0, 0, 0)),
            pl.BlockSpec((_BS, K), lambda i: (i, 0)),
            pl.BlockSpec((_BS, K, K), lambda i: (i, 0, 0)),
            pl.BlockSpec((D, D), lambda i: (0, 0)),
            pl.BlockSpec((1, D), lambda i: (0, 0)),
            pl.BlockSpec((D, D), lambda i: (0, 0)),
            pl.BlockSpec((1, D), lambda i: (0, 0)),
        ],
        out_specs=pl.BlockSpec((_BS, D), lambda i: (i, 0)),
        out_shape=jax.ShapeDtypeStruct((B, D), _f32),
    )(ev0, ev1, ev2, s0, s1, W0, b0, W1, b1)


def kernel(users, items, adj_entity, adj_relation, user_emb, entity_emb,
           relation_emb, W0, b0, W1, b1):
    users = users.astype(_i32)
    items = items.astype(_i32)
    adj_entity = adj_entity.astype(_i32)
    adj_relation = adj_relation.astype(_i32)

    u, ev0 = _k1(users, items, user_emb, entity_emb)

    rel_pad = jnp.zeros((NR_PAD, D), _f32).at[:relation_emb.shape[0]].set(relation_emb)
    p = _k2(u, rel_pad)

    ev1, e2, s1, s0 = _k3a(p, items, adj_entity, adj_relation, entity_emb)
    ev2 = _k3b(e2.reshape(-1), entity_emb)

    item = _k4(ev0, ev1, ev2.reshape(B, K, K, D), s0, s1,
               W0, b0.reshape(1, D), W1, b1.reshape(1, D))
    return (u, item[:, None, :])


# speedup vs baseline: 10.0747x; 1.3970x over previous
import jax
import jax.numpy as jnp
from jax import lax
from jax.experimental import pallas as pl
from jax.experimental.pallas import tpu as pltpu
from jax.experimental.pallas import tpu_sc as plsc

B = 4096
D = 32
K = 16
NR_PAD = 128

NC = 2
NS = 16
NW = NC * NS
NB = B // NW

_f32 = jnp.float32
_i32 = jnp.int32


def _mesh():
    return plsc.VectorSubcoreMesh(
        core_axis_name="c", subcore_axis_name="s", num_cores=NC, num_subcores=NS)


_SC_PARAMS = pltpu.CompilerParams(use_tc_tiling_on_sc=False,
                                  needs_layout_passes=False)


def _wid():
    return lax.axis_index("s") * NC + lax.axis_index("c")


def _k1_body(users_hbm, items_hbm, uemb_hbm, eemb_hbm, adje_hbm, adjr_hbm,
             u_out, ev0_out, e1_out, r0_out,
             uidx_v, iidx_v, u_v, e_v, e1_v, r0_v, sem):
    base = _wid() * NB
    pltpu.sync_copy(users_hbm.at[pl.ds(base, NB)], uidx_v)
    pltpu.sync_copy(items_hbm.at[pl.ds(base, NB)], iidx_v)
    c1 = pltpu.async_copy(uemb_hbm.at[uidx_v], u_v, sem)
    c2 = pltpu.async_copy(eemb_hbm.at[iidx_v], e_v, sem)
    c3 = pltpu.async_copy(adje_hbm.at[iidx_v], e1_v, sem)
    c4 = pltpu.async_copy(adjr_hbm.at[iidx_v], r0_v, sem)
    c1.wait()
    c2.wait()
    c3.wait()
    c4.wait()
    pltpu.sync_copy(u_v, u_out.at[pl.ds(base, NB)])
    pltpu.sync_copy(e_v, ev0_out.at[pl.ds(base, NB)])
    pltpu.sync_copy(e1_v, e1_out.at[pl.ds(base, NB)])
    pltpu.sync_copy(r0_v, r0_out.at[pl.ds(base, NB)])


def _k1(users, items, user_emb, entity_emb, adj_entity, adj_relation):
    fn = pl.kernel(
        _k1_body,
        out_type=(jax.ShapeDtypeStruct((B, D), _f32),
                  jax.ShapeDtypeStruct((B, D), _f32),
                  jax.ShapeDtypeStruct((B, K), _i32),
                  jax.ShapeDtypeStruct((B, K), _i32)),
        mesh=_mesh(),
        compiler_params=_SC_PARAMS,
        scratch_types=[
            pltpu.VMEM((NB,), _i32),
            pltpu.VMEM((NB,), _i32),
            pltpu.VMEM((NB, D), _f32),
            pltpu.VMEM((NB, D), _f32),
            pltpu.VMEM((NB, K), _i32),
            pltpu.VMEM((NB, K), _i32),
            pltpu.SemaphoreType.DMA,
        ],
    )
    return fn(users, items, user_emb, entity_emb, adj_entity, adj_relation)


def _k2_body(u_ref, r_ref, o_ref):
    o_ref[...] = lax.dot_general(
        u_ref[...], r_ref[...], (((1,), (1,)), ((), ())),
        preferred_element_type=_f32) * (1.0 / D)


def _k2(u, rel_pad):
    return pl.pallas_call(
        _k2_body,
        out_shape=jax.ShapeDtypeStruct((B, NR_PAD), _f32),
    )(u, rel_pad)


_H1 = B * K // NW
_CH = 128
_NCH3 = _H1 // _CH
_R3 = 3


def _k3_body(e1f_hbm, p_hbm, r0_hbm, adje_hbm, adjr_hbm, eemb_hbm,
             e2_out, ev1_out, s1_out, s0_out,
             e1f_v, p_v, r0_v, s0_v, e2_b, r1_b, ev1_b, s1_b,
             gsems, osems):
    base = _wid() * NB
    rbase = _wid() * _H1
    pltpu.sync_copy(e1f_hbm.at[pl.ds(rbase, _H1)], e1f_v)
    pltpu.sync_copy(p_hbm.at[pl.ds(base, NB)], p_v)
    pltpu.sync_copy(r0_hbm.at[pl.ds(base, NB)], r0_v)

    def issue(c, r):
        idx = e1f_v.at[pl.ds(c * _CH, _CH)]
        pltpu.async_copy(adje_hbm.at[idx], e2_b[r], gsems[r])
        pltpu.async_copy(adjr_hbm.at[idx], r1_b[r], gsems[r])
        pltpu.async_copy(eemb_hbm.at[idx], ev1_b[r], gsems[r])

    def wait_g(r):
        pltpu.make_async_copy(adje_hbm.at[pl.ds(0, _CH)], e2_b[r], gsems[r]).wait()
        pltpu.make_async_copy(adjr_hbm.at[pl.ds(0, _CH)], r1_b[r], gsems[r]).wait()
        pltpu.make_async_copy(eemb_hbm.at[pl.ds(0, _CH)], ev1_b[r], gsems[r]).wait()

    def wait_o(r):
        pltpu.make_async_copy(e2_b[r], e2_out.at[pl.ds(0, _CH)], osems[r]).wait()
        pltpu.make_async_copy(ev1_b[r], ev1_out.at[pl.ds(0, _CH)], osems[r]).wait()
        pltpu.make_async_copy(s1_b[r], s1_out.at[pl.ds(0, _CH)], osems[r]).wait()

    issue(0, 0)
    issue(1, 1)
    for c in range(_NCH3):
        r = c % _R3
        if c + 2 < _NCH3:
            r2 = (c + 2) % _R3
            if c + 2 - _R3 >= 0:
                wait_o(r2)
            issue(c + 2, r2)
        wait_g(r)

        def score(t, carry):
            bloc = (c * _CH + t) >> 4
            bvec = jnp.full((K,), bloc, _i32)
            s1_b[r][t] = plsc.load_gather(p_v, [bvec, r1_b[r][t]])
            return carry

        lax.fori_loop(0, _CH, score, 0)

        off = rbase + c * _CH
        pltpu.async_copy(e2_b[r], e2_out.at[pl.ds(off, _CH)], osems[r])
        pltpu.async_copy(ev1_b[r], ev1_out.at[pl.ds(off, _CH)], osems[r])
        pltpu.async_copy(s1_b[r], s1_out.at[pl.ds(off, _CH)], osems[r])

    def score0(b, carry):
        bvec = jnp.full((K,), b, _i32)
        s0_v[b] = plsc.load_gather(p_v, [bvec, r0_v[b]])
        return carry

    lax.fori_loop(0, NB, score0, 0)
    pltpu.sync_copy(s0_v, s0_out.at[pl.ds(base, NB)])

    for c in range(_NCH3 - _R3, _NCH3):
        wait_o(c % _R3)


def _k3(e1f, p, r0, adj_entity, adj_relation, entity_emb):
    fn = pl.kernel(
        _k3_body,
        out_type=(jax.ShapeDtypeStruct((B * K, K), _i32),
                  jax.ShapeDtypeStruct((B * K, D), _f32),
                  jax.ShapeDtypeStruct((B * K, K), _f32),
                  jax.ShapeDtypeStruct((B, K), _f32)),
        mesh=_mesh(),
        compiler_params=_SC_PARAMS,
        scratch_types=[
            pltpu.VMEM((_H1,), _i32),
            pltpu.VMEM((NB, NR_PAD), _f32),
            pltpu.VMEM((NB, K), _i32),
            pltpu.VMEM((NB, K), _f32),
            [pltpu.VMEM((_CH, K), _i32)] * _R3,
            [pltpu.VMEM((_CH, K), _i32)] * _R3,
            [pltpu.VMEM((_CH, D), _f32)] * _R3,
            [pltpu.VMEM((_CH, K), _f32)] * _R3,
            [pltpu.SemaphoreType.DMA] * _R3,
            [pltpu.SemaphoreType.DMA] * _R3,
        ],
    )
    return fn(e1f, p, r0, adj_entity, adj_relation, entity_emb)


_RPT = B * K * K // NW
_NCHB = _RPT // _CH
_RB = 4


def _k3b_body(idx_hbm, eemb_hbm, out_hbm, idx_v, rows_b, gsems, osems):
    rbase = _wid() * _RPT
    pltpu.sync_copy(idx_hbm.at[pl.ds(rbase, _RPT)], idx_v)

    def issue(c, r):
        pltpu.async_copy(eemb_hbm.at[idx_v.at[pl.ds(c * _CH, _CH)]],
                         rows_b[r], gsems[r])

    def wait_g(r):
        pltpu.make_async_copy(eemb_hbm.at[pl.ds(0, _CH)], rows_b[r],
                              gsems[r]).wait()

    def wait_o(r):
        pltpu.make_async_copy(rows_b[r], out_hbm.at[pl.ds(0, _CH)],
                              osems[r]).wait()

    issue(0, 0)
    issue(1, 1)

    def body(i, carry):
        for u in range(4):
            c = i * 4 + u
            cg = c + 2
            rg = (u + 2) % _RB
            if u < 2:
                @pl.when(i > 0)
                def _():
                    wait_o(rg)
                issue(cg, rg)
            else:
                @pl.when(i < (_NCHB // 4) - 1)
                def _():
                    wait_o(rg)
                    issue(cg, rg)
            wait_g(u)
            pltpu.async_copy(rows_b[u],
                             out_hbm.at[pl.ds(rbase + c * _CH, _CH)], osems[u])
        return carry

    lax.fori_loop(0, _NCHB // 4, body, 0)
    for u in range(4):
        wait_o(u)


def _k3b(e2_flat, entity_emb):
    fn = pl.kernel(
        _k3b_body,
        out_type=jax.ShapeDtypeStruct((B * K * K, D), _f32),
        mesh=_mesh(),
        compiler_params=_SC_PARAMS,
        scratch_types=[
            pltpu.VMEM((_RPT,), _i32),
            [pltpu.VMEM((_CH, D), _f32)] * _RB,
            [pltpu.SemaphoreType.DMA] * _RB,
            [pltpu.SemaphoreType.DMA] * _RB,
        ],
    )
    return fn(e2_flat, entity_emb)


_BS = 128


def _k4_body(ev0_ref, ev1_ref, ev2_ref, s0_ref, s1_ref,
             w0_ref, c0_ref, w1_ref, c1_ref, out_ref):
    ev0 = ev0_ref[...]
    ev1 = ev1_ref[...]
    ev2 = ev2_ref[...]
    W0 = w0_ref[...]
    b0 = c0_ref[...]
    W1 = w1_ref[...]
    b1 = c1_ref[...]

    wt1 = jax.nn.softmax(s1_ref[...], axis=-1)
    agg1 = jnp.sum(wt1[..., None] * ev2, axis=2)
    x1 = (ev1 + agg1).reshape(_BS * K, D)
    h1 = jax.nn.relu(
        lax.dot_general(x1, W0, (((1,), (1,)), ((), ())),
                        preferred_element_type=_f32) + b0).reshape(_BS, K, D)

    wt0 = jax.nn.softmax(s0_ref[...], axis=-1)
    agg0 = jnp.sum(wt0[..., None] * ev1, axis=1)
    h0 = jax.nn.relu(
        lax.dot_general(ev0 + agg0, W0, (((1,), (1,)), ((), ())),
                        preferred_element_type=_f32) + b0)

    aggf = jnp.sum(wt0[..., None] * h1, axis=1)
    out_ref[...] = jnp.tanh(
        lax.dot_general(h0 + aggf, W1, (((1,), (1,)), ((), ())),
                        preferred_element_type=_f32) + b1)


def _k4(ev0, ev1, ev2, s0, s1, W0, b0, W1, b1):
    nblk = B // _BS
    return pl.pallas_call(
        _k4_body,
        grid=(nblk,),
        in_specs=[
            pl.BlockSpec((_BS, D), lambda i: (i, 0)),
            pl.BlockSpec((_BS, K, D), lambda i: (i, 0, 0)),
            pl.BlockSpec((_BS, K, K, D), lambda i: (i, 0, 0, 0)),
            pl.BlockSpec((_BS, K), lambda i: (i, 0)),
            pl.BlockSpec((_BS, K, K), lambda i: (i, 0, 0)),
            pl.BlockSpec((D, D), lambda i: (0, 0)),
            pl.BlockSpec((1, D), lambda i: (0, 0)),
            pl.BlockSpec((D, D), lambda i: (0, 0)),
            pl.BlockSpec((1, D), lambda i: (0, 0)),
        ],
        out_specs=pl.BlockSpec((_BS, D), lambda i: (i, 0)),
        out_shape=jax.ShapeDtypeStruct((B, D), _f32),
    )(ev0, ev1, ev2, s0, s1, W0, b0, W1, b1)


def kernel(users, items, adj_entity, adj_relation, user_emb, entity_emb,
           relation_emb, W0, b0, W1, b1):
    users = users.astype(_i32)
    items = items.astype(_i32)
    adj_entity = adj_entity.astype(_i32)
    adj_relation = adj_relation.astype(_i32)

    u, ev0, e1, r0 = _k1(users, items, user_emb, entity_emb,
                         adj_entity, adj_relation)

    rel_pad = jnp.zeros((NR_PAD, D), _f32).at[:relation_emb.shape[0]].set(relation_emb)
    p = _k2(u, rel_pad)

    e2, ev1, s1, s0 = _k3(e1.reshape(-1), p, r0,
                          adj_entity, adj_relation, entity_emb)
    ev2 = _k3b(e2.reshape(-1), entity_emb)

    item = _k4(ev0, ev1.reshape(B, K, D), ev2.reshape(B, K, K, D),
               s0, s1.reshape(B, K, K),
               W0, b0.reshape(1, D), W1, b1.reshape(1, D))
    return (u, item[:, None, :])


# speedup vs baseline: 15.3648x; 1.5251x over previous
import jax
import jax.numpy as jnp
from jax import lax
from jax.experimental import pallas as pl
from jax.experimental.pallas import tpu as pltpu
from jax.experimental.pallas import tpu_sc as plsc

B = 4096
D = 32
K = 16
NR_PAD = 128

NC = 2
NS = 16
NW = NC * NS
NB = B // NW

_f32 = jnp.float32
_i32 = jnp.int32


def _mesh():
    return plsc.VectorSubcoreMesh(
        core_axis_name="c", subcore_axis_name="s", num_cores=NC, num_subcores=NS)


_SC_PARAMS = pltpu.CompilerParams(use_tc_tiling_on_sc=False,
                                  needs_layout_passes=False)


def _wid():
    return lax.axis_index("s") * NC + lax.axis_index("c")


def _k1_body(users_hbm, items_hbm, uemb_hbm, eemb_hbm, adje_hbm, adjr_hbm,
             u_out, ev0_out, e1_out, r0_out,
             uidx_v, iidx_v, u_v, e_v, e1_v, r0_v, sem):
    base = _wid() * NB
    pltpu.sync_copy(users_hbm.at[pl.ds(base, NB)], uidx_v)
    pltpu.sync_copy(items_hbm.at[pl.ds(base, NB)], iidx_v)
    c1 = pltpu.async_copy(uemb_hbm.at[uidx_v], u_v, sem)
    c2 = pltpu.async_copy(eemb_hbm.at[iidx_v], e_v, sem)
    c3 = pltpu.async_copy(adje_hbm.at[iidx_v], e1_v, sem)
    c4 = pltpu.async_copy(adjr_hbm.at[iidx_v], r0_v, sem)
    c1.wait()
    c2.wait()
    c3.wait()
    c4.wait()
    pltpu.sync_copy(u_v, u_out.at[pl.ds(base, NB)])
    pltpu.sync_copy(e_v, ev0_out.at[pl.ds(base, NB)])
    pltpu.sync_copy(e1_v, e1_out.at[pl.ds(base, NB)])
    pltpu.sync_copy(r0_v, r0_out.at[pl.ds(base, NB)])


def _k1(users, items, user_emb, entity_emb, adj_entity, adj_relation):
    fn = pl.kernel(
        _k1_body,
        out_type=(jax.ShapeDtypeStruct((B, D), _f32),
                  jax.ShapeDtypeStruct((B, D), _f32),
                  jax.ShapeDtypeStruct((B, K), _i32),
                  jax.ShapeDtypeStruct((B, K), _i32)),
        mesh=_mesh(),
        compiler_params=_SC_PARAMS,
        scratch_types=[
            pltpu.VMEM((NB,), _i32),
            pltpu.VMEM((NB,), _i32),
            pltpu.VMEM((NB, D), _f32),
            pltpu.VMEM((NB, D), _f32),
            pltpu.VMEM((NB, K), _i32),
            pltpu.VMEM((NB, K), _i32),
            pltpu.SemaphoreType.DMA,
        ],
    )
    return fn(users, items, user_emb, entity_emb, adj_entity, adj_relation)


def _k2_body(u_ref, r_ref, o_ref):
    o_ref[...] = lax.dot_general(
        u_ref[...], r_ref[...], (((1,), (1,)), ((), ())),
        preferred_element_type=_f32) * (1.0 / D)


def _k2(u, rel_pad):
    return pl.pallas_call(
        _k2_body,
        out_shape=jax.ShapeDtypeStruct((B, NR_PAD), _f32),
    )(u, rel_pad)


_H1 = B * K // NW
_CH = 128
_NCH3 = _H1 // _CH
_R3 = 3


def _k3_body(e1f_hbm, p_hbm, r0_hbm, adje_hbm, adjr_hbm, eemb_hbm,
             e2_out, ev1_out, s1_out, s0_out,
             e1f_v, p_v, r0_v, s0_v, e2_b, r1_b, ev1_b, s1_b,
             gsems, osems):
    base = _wid() * NB
    rbase = _wid() * _H1
    pltpu.sync_copy(e1f_hbm.at[pl.ds(rbase, _H1)], e1f_v)
    pltpu.sync_copy(p_hbm.at[pl.ds(base, NB)], p_v)
    pltpu.sync_copy(r0_hbm.at[pl.ds(base, NB)], r0_v)

    def issue(c, r):
        idx = e1f_v.at[pl.ds(c * _CH, _CH)]
        pltpu.async_copy(adje_hbm.at[idx], e2_b[r], gsems[r])
        pltpu.async_copy(adjr_hbm.at[idx], r1_b[r], gsems[r])
        pltpu.async_copy(eemb_hbm.at[idx], ev1_b[r], gsems[r])

    def wait_g(r):
        pltpu.make_async_copy(adje_hbm.at[pl.ds(0, _CH)], e2_b[r], gsems[r]).wait()
        pltpu.make_async_copy(adjr_hbm.at[pl.ds(0, _CH)], r1_b[r], gsems[r]).wait()
        pltpu.make_async_copy(eemb_hbm.at[pl.ds(0, _CH)], ev1_b[r], gsems[r]).wait()

    def wait_o(r):
        pltpu.make_async_copy(e2_b[r], e2_out.at[pl.ds(0, _CH)], osems[r]).wait()
        pltpu.make_async_copy(ev1_b[r], ev1_out.at[pl.ds(0, _CH)], osems[r]).wait()
        pltpu.make_async_copy(s1_b[r], s1_out.at[pl.ds(0, _CH)], osems[r]).wait()

    issue(0, 0)
    issue(1, 1)
    for c in range(_NCH3):
        r = c % _R3
        if c + 2 < _NCH3:
            r2 = (c + 2) % _R3
            if c + 2 - _R3 >= 0:
                wait_o(r2)
            issue(c + 2, r2)
        wait_g(r)

        def score(t, carry):
            bloc = (c * _CH + t) >> 4
            bvec = jnp.full((K,), bloc, _i32)
            s1_b[r][t] = plsc.load_gather(p_v, [bvec, r1_b[r][t]])
            return carry

        lax.fori_loop(0, _CH, score, 0)

        off = rbase + c * _CH
        pltpu.async_copy(e2_b[r], e2_out.at[pl.ds(off, _CH)], osems[r])
        pltpu.async_copy(ev1_b[r], ev1_out.at[pl.ds(off, _CH)], osems[r])
        pltpu.async_copy(s1_b[r], s1_out.at[pl.ds(off, _CH)], osems[r])

    def score0(b, carry):
        bvec = jnp.full((K,), b, _i32)
        s0_v[b] = plsc.load_gather(p_v, [bvec, r0_v[b]])
        return carry

    lax.fori_loop(0, NB, score0, 0)
    pltpu.sync_copy(s0_v, s0_out.at[pl.ds(base, NB)])

    for c in range(_NCH3 - _R3, _NCH3):
        wait_o(c % _R3)


def _k3(e1f, p, r0, adj_entity, adj_relation, entity_emb):
    fn = pl.kernel(
        _k3_body,
        out_type=(jax.ShapeDtypeStruct((B * K, K), _i32),
                  jax.ShapeDtypeStruct((B * K, D), _f32),
                  jax.ShapeDtypeStruct((B * K, K), _f32),
                  jax.ShapeDtypeStruct((B, K), _f32)),
        mesh=_mesh(),
        compiler_params=_SC_PARAMS,
        scratch_types=[
            pltpu.VMEM((_H1,), _i32),
            pltpu.VMEM((NB, NR_PAD), _f32),
            pltpu.VMEM((NB, K), _i32),
            pltpu.VMEM((NB, K), _f32),
            [pltpu.VMEM((_CH, K), _i32)] * _R3,
            [pltpu.VMEM((_CH, K), _i32)] * _R3,
            [pltpu.VMEM((_CH, D), _f32)] * _R3,
            [pltpu.VMEM((_CH, K), _f32)] * _R3,
            [pltpu.SemaphoreType.DMA] * _R3,
            [pltpu.SemaphoreType.DMA] * _R3,
        ],
    )
    return fn(e1f, p, r0, adj_entity, adj_relation, entity_emb)


_RPT = B * K * K // NW
_NCHB = _RPT // _CH
_RB = 4


def _k3b_body(idx_hbm, eemb_hbm, out_hbm, idx_v, rows_b, gsems, osems):
    rbase = _wid() * _RPT
    pltpu.sync_copy(idx_hbm.at[pl.ds(rbase, _RPT)], idx_v)

    def issue(c, r):
        pltpu.async_copy(eemb_hbm.at[idx_v.at[pl.ds(c * _CH, _CH)]],
                         rows_b[r], gsems[r])

    def wait_g(r):
        pltpu.make_async_copy(eemb_hbm.at[pl.ds(0, _CH)], rows_b[r],
                              gsems[r]).wait()

    def wait_o(r):
        pltpu.make_async_copy(rows_b[r], out_hbm.at[pl.ds(0, _CH)],
                              osems[r]).wait()

    issue(0, 0)
    issue(1, 1)

    def body(i, carry):
        for u in range(4):
            c = i * 4 + u
            cg = c + 2
            rg = (u + 2) % _RB
            if u < 2:
                @pl.when(i > 0)
                def _():
                    wait_o(rg)
                issue(cg, rg)
            else:
                @pl.when(i < (_NCHB // 4) - 1)
                def _():
                    wait_o(rg)
                    issue(cg, rg)
            wait_g(u)
            pltpu.async_copy(rows_b[u],
                             out_hbm.at[pl.ds(rbase + c * _CH, _CH)], osems[u])
        return carry

    lax.fori_loop(0, _NCHB // 4, body, 0)
    for u in range(4):
        wait_o(u)


def _k3b(e2_flat, entity_emb):
    fn = pl.kernel(
        _k3b_body,
        out_type=jax.ShapeDtypeStruct((B * K * K, D), _f32),
        mesh=_mesh(),
        compiler_params=_SC_PARAMS,
        scratch_types=[
            pltpu.VMEM((_RPT,), _i32),
            [pltpu.VMEM((_CH, D), _f32)] * _RB,
            [pltpu.SemaphoreType.DMA] * _RB,
            [pltpu.SemaphoreType.DMA] * _RB,
        ],
    )
    return fn(e2_flat, entity_emb)


_BS = 256


def _k4_body(ev0_ref, ev1a_ref, ev1b_ref, ev2_ref, s0_ref, s1_ref,
             e_ref, sm_ref, w0_ref, c0_ref, w1_ref, c1_ref, out_ref):
    W0 = w0_ref[...]
    b0 = c0_ref[...]
    W1 = w1_ref[...]
    b1 = c1_ref[...]

    wt1 = jax.nn.softmax(s1_ref[...], axis=-1)
    wexp = jnp.dot(wt1, e_ref[...], preferred_element_type=_f32)
    agg1 = jnp.dot(ev2_ref[...] * wexp, sm_ref[...],
                   preferred_element_type=_f32)
    h1 = jax.nn.relu(
        lax.dot_general(ev1a_ref[...] + agg1, W0, (((1,), (1,)), ((), ())),
                        preferred_element_type=_f32) + b0)

    wt0 = jax.nn.softmax(s0_ref[...], axis=-1)
    agg0 = jnp.sum(wt0[..., None] * ev1b_ref[...], axis=1)
    h0 = jax.nn.relu(
        lax.dot_general(ev0_ref[...] + agg0, W0, (((1,), (1,)), ((), ())),
                        preferred_element_type=_f32) + b0)

    aggf = jnp.sum(wt0[..., None] * h1.reshape(_BS, K, D), axis=1)
    out_ref[...] = jnp.tanh(
        lax.dot_general(h0 + aggf, W1, (((1,), (1,)), ((), ())),
                        preferred_element_type=_f32) + b1)


def _k4(ev0, ev1a, ev1b, ev2, s0, s1, emat, smat, W0, b0, W1, b1):
    nblk = B // _BS
    return pl.pallas_call(
        _k4_body,
        grid=(nblk,),
        in_specs=[
            pl.BlockSpec((_BS, D), lambda i: (i, 0)),
            pl.BlockSpec((_BS * K, D), lambda i: (i, 0)),
            pl.BlockSpec((_BS, K, D), lambda i: (i, 0, 0)),
            pl.BlockSpec((_BS * K, K * D), lambda i: (i, 0)),
            pl.BlockSpec((_BS, K), lambda i: (i, 0)),
            pl.BlockSpec((_BS * K, K), lambda i: (i, 0)),
            pl.BlockSpec((K, K * D), lambda i: (0, 0)),
            pl.BlockSpec((K * D, D), lambda i: (0, 0)),
            pl.BlockSpec((D, D), lambda i: (0, 0)),
            pl.BlockSpec((1, D), lambda i: (0, 0)),
            pl.BlockSpec((D, D), lambda i: (0, 0)),
            pl.BlockSpec((1, D), lambda i: (0, 0)),
        ],
        out_specs=pl.BlockSpec((_BS, D), lambda i: (i, 0)),
        out_shape=jax.ShapeDtypeStruct((B, D), _f32),
    )(ev0, ev1a, ev1b, ev2, s0, s1, emat, smat, W0, b0, W1, b1)


def kernel(users, items, adj_entity, adj_relation, user_emb, entity_emb,
           relation_emb, W0, b0, W1, b1):
    users = users.astype(_i32)
    items = items.astype(_i32)
    adj_entity = adj_entity.astype(_i32)
    adj_relation = adj_relation.astype(_i32)

    u, ev0, e1, r0 = _k1(users, items, user_emb, entity_emb,
                         adj_entity, adj_relation)

    rel_pad = jnp.zeros((NR_PAD, D), _f32).at[:relation_emb.shape[0]].set(relation_emb)
    p = _k2(u, rel_pad)

    e2, ev1, s1, s0 = _k3(e1.reshape(-1), p, r0,
                          adj_entity, adj_relation, entity_emb)
    ev2 = _k3b(e2.reshape(-1), entity_emb)

    emat = jnp.kron(jnp.eye(K, dtype=_f32), jnp.ones((1, D), _f32))
    smat = jnp.kron(jnp.ones((K, 1), _f32), jnp.eye(D, dtype=_f32))
    item = _k4(ev0, ev1, ev1.reshape(B, K, D), ev2.reshape(B * K, K * D),
               s0, s1, emat, smat,
               W0, b0.reshape(1, D), W1, b1.reshape(1, D))
    return (u, item[:, None, :])


# speedup vs baseline: 19.2886x; 1.2554x over previous
import jax
import jax.numpy as jnp
from jax import lax
from jax.experimental import pallas as pl
from jax.experimental.pallas import tpu as pltpu
from jax.experimental.pallas import tpu_sc as plsc

B = 4096
D = 32
K = 16
NR_PAD = 128

NC = 2
NS = 16
NW = NC * NS
NB = B // NW

_f32 = jnp.float32
_i32 = jnp.int32


def _mesh():
    return plsc.VectorSubcoreMesh(
        core_axis_name="c", subcore_axis_name="s", num_cores=NC, num_subcores=NS)


_SC_PARAMS = pltpu.CompilerParams(use_tc_tiling_on_sc=False,
                                  needs_layout_passes=False)


def _wid():
    return lax.axis_index("s") * NC + lax.axis_index("c")


def _k1_body(users_hbm, items_hbm, uemb_hbm, eemb_hbm, adje_hbm, adjr_hbm,
             u_out, ev0_out, e1_out, r0_out,
             uidx_v, iidx_v, u_v, e_v, e1_v, r0_v, sem):
    base = _wid() * NB
    pltpu.sync_copy(users_hbm.at[pl.ds(base, NB)], uidx_v)
    pltpu.sync_copy(items_hbm.at[pl.ds(base, NB)], iidx_v)
    c1 = pltpu.async_copy(uemb_hbm.at[uidx_v], u_v, sem)
    c2 = pltpu.async_copy(eemb_hbm.at[iidx_v], e_v, sem)
    c3 = pltpu.async_copy(adje_hbm.at[iidx_v], e1_v, sem)
    c4 = pltpu.async_copy(adjr_hbm.at[iidx_v], r0_v, sem)
    c1.wait()
    c2.wait()
    c3.wait()
    c4.wait()
    pltpu.sync_copy(u_v, u_out.at[pl.ds(base, NB)])
    pltpu.sync_copy(e_v, ev0_out.at[pl.ds(base, NB)])
    pltpu.sync_copy(e1_v, e1_out.at[pl.ds(base, NB)])
    pltpu.sync_copy(r0_v, r0_out.at[pl.ds(base, NB)])


def _k1(users, items, user_emb, entity_emb, adj_entity, adj_relation):
    fn = pl.kernel(
        _k1_body,
        out_type=(jax.ShapeDtypeStruct((B, D), _f32),
                  jax.ShapeDtypeStruct((B, D), _f32),
                  jax.ShapeDtypeStruct((B, K), _i32),
                  jax.ShapeDtypeStruct((B, K), _i32)),
        mesh=_mesh(),
        compiler_params=_SC_PARAMS,
        scratch_types=[
            pltpu.VMEM((NB,), _i32),
            pltpu.VMEM((NB,), _i32),
            pltpu.VMEM((NB, D), _f32),
            pltpu.VMEM((NB, D), _f32),
            pltpu.VMEM((NB, K), _i32),
            pltpu.VMEM((NB, K), _i32),
            pltpu.SemaphoreType.DMA,
        ],
    )
    return fn(users, items, user_emb, entity_emb, adj_entity, adj_relation)


def _k2_body(u_ref, r_ref, o_ref):
    o_ref[...] = lax.dot_general(
        u_ref[...], r_ref[...], (((1,), (1,)), ((), ())),
        preferred_element_type=_f32) * (1.0 / D)


def _k2(u, rel_pad):
    return pl.pallas_call(
        _k2_body,
        out_shape=jax.ShapeDtypeStruct((B, NR_PAD), _f32),
    )(u, rel_pad)


_H1 = B * K // NW
_CH = 128
_NCH3 = _H1 // _CH
_R3 = 3


def _k3_body(e1f_hbm, p_hbm, r0_hbm, adje_hbm, adjr_hbm, eemb_hbm,
             e2_out, ev1_out, s1_out, s0_out,
             e1f_v, p_v, r0_v, s0_v, e2_b, r1_b, ev1_b, s1_b,
             gsems, osems):
    base = _wid() * NB
    rbase = _wid() * _H1
    pltpu.sync_copy(e1f_hbm.at[pl.ds(rbase, _H1)], e1f_v)
    pltpu.sync_copy(p_hbm.at[pl.ds(base, NB)], p_v)
    pltpu.sync_copy(r0_hbm.at[pl.ds(base, NB)], r0_v)

    def issue(c, r):
        idx = e1f_v.at[pl.ds(c * _CH, _CH)]
        pltpu.async_copy(adje_hbm.at[idx], e2_b[r], gsems[r])
        pltpu.async_copy(adjr_hbm.at[idx], r1_b[r], gsems[r])
        pltpu.async_copy(eemb_hbm.at[idx], ev1_b[r], gsems[r])

    def wait_g(r):
        pltpu.make_async_copy(adje_hbm.at[pl.ds(0, _CH)], e2_b[r], gsems[r]).wait()
        pltpu.make_async_copy(adjr_hbm.at[pl.ds(0, _CH)], r1_b[r], gsems[r]).wait()
        pltpu.make_async_copy(eemb_hbm.at[pl.ds(0, _CH)], ev1_b[r], gsems[r]).wait()

    def wait_o(r):
        pltpu.make_async_copy(e2_b[r], e2_out.at[pl.ds(0, _CH)], osems[r]).wait()
        pltpu.make_async_copy(ev1_b[r], ev1_out.at[pl.ds(0, _CH)], osems[r]).wait()
        pltpu.make_async_copy(s1_b[r], s1_out.at[pl.ds(0, _CH)], osems[r]).wait()

    issue(0, 0)
    issue(1, 1)
    for c in range(_NCH3):
        r = c % _R3
        if c + 2 < _NCH3:
            r2 = (c + 2) % _R3
            if c + 2 - _R3 >= 0:
                wait_o(r2)
            issue(c + 2, r2)
        wait_g(r)

        def score(t, carry):
            bloc = (c * _CH + t) >> 4
            bvec = jnp.full((K,), bloc, _i32)
            s1_b[r][t] = plsc.load_gather(p_v, [bvec, r1_b[r][t]])
            return carry

        lax.fori_loop(0, _CH, score, 0)

        off = rbase + c * _CH
        pltpu.async_copy(e2_b[r], e2_out.at[pl.ds(off, _CH)], osems[r])
        pltpu.async_copy(ev1_b[r], ev1_out.at[pl.ds(off, _CH)], osems[r])
        pltpu.async_copy(s1_b[r], s1_out.at[pl.ds(off, _CH)], osems[r])

    def score0(b, carry):
        bvec = jnp.full((K,), b, _i32)
        s0_v[b] = plsc.load_gather(p_v, [bvec, r0_v[b]])
        return carry

    lax.fori_loop(0, NB, score0, 0)
    pltpu.sync_copy(s0_v, s0_out.at[pl.ds(base, NB)])

    for c in range(_NCH3 - _R3, _NCH3):
        wait_o(c % _R3)


def _k3(e1f, p, r0, adj_entity, adj_relation, entity_emb):
    fn = pl.kernel(
        _k3_body,
        out_type=(jax.ShapeDtypeStruct((B * K, K), _i32),
                  jax.ShapeDtypeStruct((B * K, D), _f32),
                  jax.ShapeDtypeStruct((B * K, K), _f32),
                  jax.ShapeDtypeStruct((B, K), _f32)),
        mesh=_mesh(),
        compiler_params=_SC_PARAMS,
        scratch_types=[
            pltpu.VMEM((_H1,), _i32),
            pltpu.VMEM((NB, NR_PAD), _f32),
            pltpu.VMEM((NB, K), _i32),
            pltpu.VMEM((NB, K), _f32),
            [pltpu.VMEM((_CH, K), _i32)] * _R3,
            [pltpu.VMEM((_CH, K), _i32)] * _R3,
            [pltpu.VMEM((_CH, D), _f32)] * _R3,
            [pltpu.VMEM((_CH, K), _f32)] * _R3,
            [pltpu.SemaphoreType.DMA] * _R3,
            [pltpu.SemaphoreType.DMA] * _R3,
        ],
    )
    return fn(e1f, p, r0, adj_entity, adj_relation, entity_emb)


_RPT = B * K * K // NW
_NCHB = _RPT // _CH
_GPC = _CH // K
_RB = 2
_HD = D // 2


def _k3b_body(idx_hbm, s1_hbm, eemb_hbm, out_hbm,
              idx_v, rows_b, s1_b, acc_b, gsems, osems):
    rbase = _wid() * _RPT
    gbase = _wid() * (B * K // NW)
    pltpu.sync_copy(idx_hbm.at[pl.ds(rbase, _RPT)], idx_v)

    def issue(c, r):
        pltpu.async_copy(eemb_hbm.at[idx_v.at[pl.ds(c * _CH, _CH)]],
                         rows_b[r], gsems[r])
        pltpu.async_copy(s1_hbm.at[pl.ds(gbase + c * _GPC, _GPC)],
                         s1_b[r], gsems[r])

    def wait_g(r):
        pltpu.make_async_copy(eemb_hbm.at[pl.ds(0, _CH)], rows_b[r],
                              gsems[r]).wait()
        pltpu.make_async_copy(s1_hbm.at[pl.ds(0, _GPC)], s1_b[r],
                              gsems[r]).wait()

    def wait_o(r):
        pltpu.make_async_copy(acc_b[r], out_hbm.at[pl.ds(0, _GPC)],
                              osems[r]).wait()

    def compute(c, r):
        def group(g, carry):
            srow = s1_b[r][g]
            e = jnp.exp(srow - jnp.max(srow))
            w = e / jnp.broadcast_to(jnp.sum(e), (K,))
            lo = jnp.zeros((K,), _f32)
            hi = jnp.zeros((K,), _f32)
            for k in range(K):
                wk = w[k]
                lo = lo + rows_b[r][g * K + k, pl.ds(0, _HD)] * wk
                hi = hi + rows_b[r][g * K + k, pl.ds(_HD, _HD)] * wk
            acc_b[r][g, pl.ds(0, _HD)] = lo
            acc_b[r][g, pl.ds(_HD, _HD)] = hi
            return carry

        lax.fori_loop(0, _GPC, group, 0)
        pltpu.async_copy(acc_b[r],
                         out_hbm.at[pl.ds(gbase + c * _GPC, _GPC)], osems[r])

    issue(0, 0)

    def body(i, carry):
        for u in range(_RB):
            c = i * _RB + u
            cg = c + 1
            rg = (u + 1) % _RB
            if u == _RB - 1:
                @pl.when(i < (_NCHB // _RB) - 1)
                def _():
                    wait_o(rg)
                    issue(cg, rg)
            else:
                @pl.when(i > 0)
                def _():
                    wait_o(rg)
                issue(cg, rg)
            wait_g(u)
            compute(c, u)
        return carry

    lax.fori_loop(0, _NCHB // _RB, body, 0)
    for u in range(_RB):
        wait_o(u)


def _k3b(e2_flat, s1, entity_emb):
    fn = pl.kernel(
        _k3b_body,
        out_type=jax.ShapeDtypeStruct((B * K, D), _f32),
        mesh=_mesh(),
        compiler_params=_SC_PARAMS,
        scratch_types=[
            pltpu.VMEM((_RPT,), _i32),
            [pltpu.VMEM((_CH, D), _f32)] * _RB,
            [pltpu.VMEM((_GPC, K), _f32)] * _RB,
            [pltpu.VMEM((_GPC, D), _f32)] * _RB,
            [pltpu.SemaphoreType.DMA] * _RB,
            [pltpu.SemaphoreType.DMA] * _RB,
        ],
    )
    return fn(e2_flat, s1, entity_emb)


_BS = 512


def _k4_body(ev0_ref, ev1a_ref, ev1b_ref, agg1_ref, s0_ref,
             w0_ref, c0_ref, w1_ref, c1_ref, out_ref):
    W0 = w0_ref[...]
    b0 = c0_ref[...]
    W1 = w1_ref[...]
    b1 = c1_ref[...]

    h1 = jax.nn.relu(
        lax.dot_general(ev1a_ref[...] + agg1_ref[...], W0,
                        (((1,), (1,)), ((), ())),
                        preferred_element_type=_f32) + b0)

    wt0 = jax.nn.softmax(s0_ref[...], axis=-1)
    agg0 = jnp.sum(wt0[..., None] * ev1b_ref[...], axis=1)
    h0 = jax.nn.relu(
        lax.dot_general(ev0_ref[...] + agg0, W0, (((1,), (1,)), ((), ())),
                        preferred_element_type=_f32) + b0)

    aggf = jnp.sum(wt0[..., None] * h1.reshape(_BS, K, D), axis=1)
    out_ref[...] = jnp.tanh(
        lax.dot_general(h0 + aggf, W1, (((1,), (1,)), ((), ())),
                        preferred_element_type=_f32) + b1)


def _k4(ev0, ev1a, ev1b, agg1, s0, W0, b0, W1, b1):
    nblk = B // _BS
    return pl.pallas_call(
        _k4_body,
        grid=(nblk,),
        in_specs=[
            pl.BlockSpec((_BS, D), lambda i: (i, 0)),
            pl.BlockSpec((_BS * K, D), lambda i: (i, 0)),
            pl.BlockSpec((_BS, K, D), lambda i: (i, 0, 0)),
            pl.BlockSpec((_BS * K, D), lambda i: (i, 0)),
            pl.BlockSpec((_BS, K), lambda i: (i, 0)),
            pl.BlockSpec((D, D), lambda i: (0, 0)),
            pl.BlockSpec((1, D), lambda i: (0, 0)),
            pl.BlockSpec((D, D), lambda i: (0, 0)),
            pl.BlockSpec((1, D), lambda i: (0, 0)),
        ],
        out_specs=pl.BlockSpec((_BS, D), lambda i: (i, 0)),
        out_shape=jax.ShapeDtypeStruct((B, D), _f32),
    )(ev0, ev1a, ev1b, agg1, s0, W0, b0, W1, b1)


def kernel(users, items, adj_entity, adj_relation, user_emb, entity_emb,
           relation_emb, W0, b0, W1, b1):
    users = users.astype(_i32)
    items = items.astype(_i32)
    adj_entity = adj_entity.astype(_i32)
    adj_relation = adj_relation.astype(_i32)

    u, ev0, e1, r0 = _k1(users, items, user_emb, entity_emb,
                         adj_entity, adj_relation)

    rel_pad = jnp.zeros((NR_PAD, D), _f32).at[:relation_emb.shape[0]].set(relation_emb)
    p = _k2(u, rel_pad)

    e2, ev1, s1, s0 = _k3(e1.reshape(-1), p, r0,
                          adj_entity, adj_relation, entity_emb)
    agg1 = _k3b(e2.reshape(-1), s1, entity_emb)

    item = _k4(ev0, ev1, ev1.reshape(B, K, D), agg1, s0,
               W0, b0.reshape(1, D), W1, b1.reshape(1, D))
    return (u, item[:, None, :])


# speedup vs baseline: 21.7577x; 1.1280x over previous
import jax
import jax.numpy as jnp
from jax import lax
from jax.experimental import pallas as pl
from jax.experimental.pallas import tpu as pltpu
from jax.experimental.pallas import tpu_sc as plsc

B = 4096
D = 32
K = 16
NR_PAD = 128

NC = 2
NS = 16
NW = NC * NS
NB = B // NW

_f32 = jnp.float32
_i32 = jnp.int32


def _mesh():
    return plsc.VectorSubcoreMesh(
        core_axis_name="c", subcore_axis_name="s", num_cores=NC, num_subcores=NS)


_SC_PARAMS = pltpu.CompilerParams(use_tc_tiling_on_sc=False,
                                  needs_layout_passes=False)


def _wid():
    return lax.axis_index("s") * NC + lax.axis_index("c")


def _k1_body(users_hbm, items_hbm, uemb_hbm, eemb_hbm, adje_hbm, adjr_hbm,
             u_out, ev0_out, e1_out, r0_out,
             uidx_v, iidx_v, u_v, e_v, e1_v, r0_v, sem):
    base = _wid() * NB
    pltpu.sync_copy(users_hbm.at[pl.ds(base, NB)], uidx_v)
    pltpu.sync_copy(items_hbm.at[pl.ds(base, NB)], iidx_v)
    c1 = pltpu.async_copy(uemb_hbm.at[uidx_v], u_v, sem)
    c2 = pltpu.async_copy(eemb_hbm.at[iidx_v], e_v, sem)
    c3 = pltpu.async_copy(adje_hbm.at[iidx_v], e1_v, sem)
    c4 = pltpu.async_copy(adjr_hbm.at[iidx_v], r0_v, sem)
    c1.wait()
    c2.wait()
    c3.wait()
    c4.wait()
    pltpu.sync_copy(u_v, u_out.at[pl.ds(base, NB)])
    pltpu.sync_copy(e_v, ev0_out.at[pl.ds(base, NB)])
    pltpu.sync_copy(e1_v, e1_out.at[pl.ds(base, NB)])
    pltpu.sync_copy(r0_v, r0_out.at[pl.ds(base, NB)])


def _k1(users, items, user_emb, entity_emb, adj_entity, adj_relation):
    fn = pl.kernel(
        _k1_body,
        out_type=(jax.ShapeDtypeStruct((B, D), _f32),
                  jax.ShapeDtypeStruct((B, D), _f32),
                  jax.ShapeDtypeStruct((B, K), _i32),
                  jax.ShapeDtypeStruct((B, K), _i32)),
        mesh=_mesh(),
        compiler_params=_SC_PARAMS,
        scratch_types=[
            pltpu.VMEM((NB,), _i32),
            pltpu.VMEM((NB,), _i32),
            pltpu.VMEM((NB, D), _f32),
            pltpu.VMEM((NB, D), _f32),
            pltpu.VMEM((NB, K), _i32),
            pltpu.VMEM((NB, K), _i32),
            pltpu.SemaphoreType.DMA,
        ],
    )
    return fn(users, items, user_emb, entity_emb, adj_entity, adj_relation)


def _k2_body(u_ref, r_ref, o_ref):
    o_ref[...] = lax.dot_general(
        u_ref[...], r_ref[...], (((1,), (1,)), ((), ())),
        preferred_element_type=_f32) * (1.0 / D)


def _k2(u, rel_pad):
    return pl.pallas_call(
        _k2_body,
        out_shape=jax.ShapeDtypeStruct((B, NR_PAD), _f32),
    )(u, rel_pad)


_H1 = B * K // NW
_CH = 128
_NCH3 = _H1 // _CH
_R3 = 3
_GPC = _CH // K
_HD = D // 2


def _k3_body(e1f_hbm, p_hbm, r0_hbm, adje_hbm, adjr_hbm, eemb_hbm,
             e2_out, ev1_out, s1_out, wt0_out, agg0_out,
             e1f_v, p_v, r0_v, wt0_v, agg0_v, e2_b, r1_b, ev1_b, s1_b,
             gsems, osems):
    base = _wid() * NB
    rbase = _wid() * _H1
    pltpu.sync_copy(e1f_hbm.at[pl.ds(rbase, _H1)], e1f_v)
    pltpu.sync_copy(p_hbm.at[pl.ds(base, NB)], p_v)
    pltpu.sync_copy(r0_hbm.at[pl.ds(base, NB)], r0_v)

    def issue(c, r):
        idx = e1f_v.at[pl.ds(c * _CH, _CH)]
        pltpu.async_copy(adje_hbm.at[idx], e2_b[r], gsems[r])
        pltpu.async_copy(adjr_hbm.at[idx], r1_b[r], gsems[r])
        pltpu.async_copy(eemb_hbm.at[idx], ev1_b[r], gsems[r])

    def wait_g(r):
        pltpu.make_async_copy(adje_hbm.at[pl.ds(0, _CH)], e2_b[r], gsems[r]).wait()
        pltpu.make_async_copy(adjr_hbm.at[pl.ds(0, _CH)], r1_b[r], gsems[r]).wait()
        pltpu.make_async_copy(eemb_hbm.at[pl.ds(0, _CH)], ev1_b[r], gsems[r]).wait()

    def wait_o(r):
        pltpu.make_async_copy(e2_b[r], e2_out.at[pl.ds(0, _CH)], osems[r]).wait()
        pltpu.make_async_copy(ev1_b[r], ev1_out.at[pl.ds(0, _CH)], osems[r]).wait()
        pltpu.make_async_copy(s1_b[r], s1_out.at[pl.ds(0, _CH)], osems[r]).wait()

    issue(0, 0)
    issue(1, 1)
    for c in range(_NCH3):
        r = c % _R3
        if c + 2 < _NCH3:
            r2 = (c + 2) % _R3
            if c + 2 - _R3 >= 0:
                wait_o(r2)
            issue(c + 2, r2)
        wait_g(r)

        def score(t, carry):
            bloc = (c * _CH + t) >> 4
            bvec = jnp.full((K,), bloc, _i32)
            s1_b[r][t] = plsc.load_gather(p_v, [bvec, r1_b[r][t]])
            return carry

        lax.fori_loop(0, _CH, score, 0)

        def hop0(g, carry):
            b = c * _GPC + g
            bvec = jnp.full((K,), b, _i32)
            srow = plsc.load_gather(p_v, [bvec, r0_v[b]])
            e = jnp.exp(srow - jnp.broadcast_to(jnp.max(srow), (K,)))
            w = e / jnp.broadcast_to(jnp.sum(e), (K,))
            wt0_v[b] = w
            lo = jnp.zeros((_HD,), _f32)
            hi = jnp.zeros((_HD,), _f32)
            for k in range(K):
                wk = w[k]
                lo = lo + ev1_b[r][g * K + k, pl.ds(0, _HD)] * wk
                hi = hi + ev1_b[r][g * K + k, pl.ds(_HD, _HD)] * wk
            agg0_v[b, pl.ds(0, _HD)] = lo
            agg0_v[b, pl.ds(_HD, _HD)] = hi
            return carry

        lax.fori_loop(0, _GPC, hop0, 0)

        off = rbase + c * _CH
        pltpu.async_copy(e2_b[r], e2_out.at[pl.ds(off, _CH)], osems[r])
        pltpu.async_copy(ev1_b[r], ev1_out.at[pl.ds(off, _CH)], osems[r])
        pltpu.async_copy(s1_b[r], s1_out.at[pl.ds(off, _CH)], osems[r])

    pltpu.sync_copy(wt0_v, wt0_out.at[pl.ds(base, NB)])
    pltpu.sync_copy(agg0_v, agg0_out.at[pl.ds(base, NB)])

    for c in range(_NCH3 - _R3, _NCH3):
        wait_o(c % _R3)


def _k3(e1f, p, r0, adj_entity, adj_relation, entity_emb):
    fn = pl.kernel(
        _k3_body,
        out_type=(jax.ShapeDtypeStruct((B * K, K), _i32),
                  jax.ShapeDtypeStruct((B * K, D), _f32),
                  jax.ShapeDtypeStruct((B * K, K), _f32),
                  jax.ShapeDtypeStruct((B, K), _f32),
                  jax.ShapeDtypeStruct((B, D), _f32)),
        mesh=_mesh(),
        compiler_params=_SC_PARAMS,
        scratch_types=[
            pltpu.VMEM((_H1,), _i32),
            pltpu.VMEM((NB, NR_PAD), _f32),
            pltpu.VMEM((NB, K), _i32),
            pltpu.VMEM((NB, K), _f32),
            pltpu.VMEM((NB, D), _f32),
            [pltpu.VMEM((_CH, K), _i32)] * _R3,
            [pltpu.VMEM((_CH, K), _i32)] * _R3,
            [pltpu.VMEM((_CH, D), _f32)] * _R3,
            [pltpu.VMEM((_CH, K), _f32)] * _R3,
            [pltpu.SemaphoreType.DMA] * _R3,
            [pltpu.SemaphoreType.DMA] * _R3,
        ],
    )
    return fn(e1f, p, r0, adj_entity, adj_relation, entity_emb)


_RPT = B * K * K // NW
_NCHB = _RPT // _CH
_RB = 2
_PK = 4


def _k3b_body(idx_hbm, s1_hbm, ev1_hbm, eemb_hbm, out_hbm,
              idx_v, rows_b, s1_b, ev1c_b, acc_b, gsems, osems):
    rbase = _wid() * _RPT
    gbase = _wid() * (B * K // NW)
    pltpu.sync_copy(idx_hbm.at[pl.ds(rbase, _RPT)], idx_v)

    def issue(c, r):
        pltpu.async_copy(eemb_hbm.at[idx_v.at[pl.ds(c * _CH, _CH)]],
                         rows_b[r], gsems[r])
        pltpu.async_copy(s1_hbm.at[pl.ds(gbase + c * _GPC, _GPC)],
                         s1_b[r], gsems[r])
        pltpu.async_copy(ev1_hbm.at[pl.ds(gbase + c * _GPC, _GPC)],
                         ev1c_b[r], gsems[r])

    def wait_g(r):
        pltpu.make_async_copy(eemb_hbm.at[pl.ds(0, _CH)], rows_b[r],
                              gsems[r]).wait()
        pltpu.make_async_copy(s1_hbm.at[pl.ds(0, _GPC)], s1_b[r],
                              gsems[r]).wait()
        pltpu.make_async_copy(ev1_hbm.at[pl.ds(0, _GPC)], ev1c_b[r],
                              gsems[r]).wait()

    def wait_o(r):
        pltpu.make_async_copy(acc_b[r], out_hbm.at[pl.ds(0, _GPC // _PK)],
                              osems[r]).wait()

    def compute(c, r):
        for g in range(_GPC):
            srow = s1_b[r][g]
            e = jnp.exp(srow - jnp.broadcast_to(jnp.max(srow), (K,)))
            w = e / jnp.broadcast_to(jnp.sum(e), (K,))
            lo = ev1c_b[r][g, pl.ds(0, _HD)]
            hi = ev1c_b[r][g, pl.ds(_HD, _HD)]
            for k in range(K):
                wk = w[k]
                lo = lo + rows_b[r][g * K + k, pl.ds(0, _HD)] * wk
                hi = hi + rows_b[r][g * K + k, pl.ds(_HD, _HD)] * wk
            qoff = (g % _PK) * D
            acc_b[r][g // _PK, pl.ds(qoff, _HD)] = lo
            acc_b[r][g // _PK, pl.ds(qoff + _HD, _HD)] = hi
        pltpu.async_copy(
            acc_b[r],
            out_hbm.at[pl.ds((gbase + c * _GPC) // _PK, _GPC // _PK)],
            osems[r])

    issue(0, 0)

    def body(i, carry):
        for u in range(_RB):
            c = i * _RB + u
            cg = c + 1
            rg = (u + 1) % _RB
            if u == _RB - 1:
                @pl.when(i < (_NCHB // _RB) - 1)
                def _():
                    wait_o(rg)
                    issue(cg, rg)
            else:
                @pl.when(i > 0)
                def _():
                    wait_o(rg)
                issue(cg, rg)
            wait_g(u)
            compute(c, u)
        return carry

    lax.fori_loop(0, _NCHB // _RB, body, 0)
    for u in range(_RB):
        wait_o(u)


def _k3b(e2_flat, s1, ev1, entity_emb):
    fn = pl.kernel(
        _k3b_body,
        out_type=jax.ShapeDtypeStruct((B * K // _PK, _PK * D), _f32),
        mesh=_mesh(),
        compiler_params=_SC_PARAMS,
        scratch_types=[
            pltpu.VMEM((_RPT,), _i32),
            [pltpu.VMEM((_CH, D), _f32)] * _RB,
            [pltpu.VMEM((_GPC, K), _f32)] * _RB,
            [pltpu.VMEM((_GPC, D), _f32)] * _RB,
            [pltpu.VMEM((_GPC // _PK, _PK * D), _f32)] * _RB,
            [pltpu.SemaphoreType.DMA] * _RB,
            [pltpu.SemaphoreType.DMA] * _RB,
        ],
    )
    return fn(e2_flat, s1, ev1, entity_emb)


_BS = 512


def _k4_body(ev0_ref, agg0_ref, wt0_ref, x1p_ref, a4_ref, bsel_ref, s4_ref,
             bw0_ref, c0t4_ref, w0_ref, c0_ref, w1_ref, c1_ref, out_ref):
    h1p = jax.nn.relu(
        jnp.dot(x1p_ref[...], bw0_ref[...], preferred_element_type=_f32)
        + c0t4_ref[...])

    w4pre = jnp.dot(a4_ref[...], wt0_ref[...],
                    preferred_element_type=_f32)
    rowq = lax.broadcasted_iota(_i32, (_BS * _PK, _PK * D), 0) % _PK
    wexp = jnp.zeros((_BS * _PK, _PK * D), _f32)
    for m in range(_PK):
        wm = jnp.dot(w4pre, bsel_ref[...][m], preferred_element_type=_f32)
        wexp = jnp.where(rowq == m, wm, wexp)
    y = (h1p * wexp).reshape(_BS, 4, _PK * D).sum(axis=1)
    aggf = jnp.dot(y, s4_ref[...], preferred_element_type=_f32)

    h0 = jax.nn.relu(
        lax.dot_general(ev0_ref[...] + agg0_ref[...], w0_ref[...],
                        (((1,), (1,)), ((), ())),
                        preferred_element_type=_f32) + c0_ref[...])

    out_ref[...] = jnp.tanh(
        lax.dot_general(h0 + aggf, w1_ref[...], (((1,), (1,)), ((), ())),
                        preferred_element_type=_f32) + c1_ref[...])


def _k4(ev0, agg0, wt0, x1p, a4, bsel, s4, bw0, b0t4, W0, b0, W1, b1):
    nblk = B // _BS
    return pl.pallas_call(
        _k4_body,
        grid=(nblk,),
        in_specs=[
            pl.BlockSpec((_BS, D), lambda i: (i, 0)),
            pl.BlockSpec((_BS, D), lambda i: (i, 0)),
            pl.BlockSpec((_BS, K), lambda i: (i, 0)),
            pl.BlockSpec((_BS * K // _PK, _PK * D), lambda i: (i, 0)),
            pl.BlockSpec((_BS * _PK, _BS), lambda i: (0, 0)),
            pl.BlockSpec((_PK, K, _PK * D), lambda i: (0, 0, 0)),
            pl.BlockSpec((_PK * D, D), lambda i: (0, 0)),
            pl.BlockSpec((_PK * D, _PK * D), lambda i: (0, 0)),
            pl.BlockSpec((1, _PK * D), lambda i: (0, 0)),
            pl.BlockSpec((D, D), lambda i: (0, 0)),
            pl.BlockSpec((1, D), lambda i: (0, 0)),
            pl.BlockSpec((D, D), lambda i: (0, 0)),
            pl.BlockSpec((1, D), lambda i: (0, 0)),
        ],
        out_specs=pl.BlockSpec((_BS, D), lambda i: (i, 0)),
        out_shape=jax.ShapeDtypeStruct((B, D), _f32),
    )(ev0, agg0, wt0, x1p, a4, bsel, s4, bw0, b0t4, W0, b0, W1, b1)


def kernel(users, items, adj_entity, adj_relation, user_emb, entity_emb,
           relation_emb, W0, b0, W1, b1):
    users = users.astype(_i32)
    items = items.astype(_i32)
    adj_entity = adj_entity.astype(_i32)
    adj_relation = adj_relation.astype(_i32)

    u, ev0, e1, r0 = _k1(users, items, user_emb, entity_emb,
                         adj_entity, adj_relation)

    rel_pad = jnp.zeros((NR_PAD, D), _f32).at[:relation_emb.shape[0]].set(relation_emb)
    p = _k2(u, rel_pad)

    e2, ev1, s1, wt0, agg0 = _k3(e1.reshape(-1), p, r0,
                                 adj_entity, adj_relation, entity_emb)
    x1p = _k3b(e2.reshape(-1), s1, ev1, entity_emb)

    s4 = jnp.kron(jnp.ones((_PK, 1), _f32), jnp.eye(D, dtype=_f32))
    bw0 = jnp.kron(jnp.eye(_PK, dtype=_f32), W0.T)
    b0t4 = jnp.tile(b0.reshape(1, D), (1, _PK))
    a4 = jnp.kron(jnp.eye(_BS, dtype=_f32), jnp.ones((_PK, 1), _f32))
    qidx = jnp.arange(_PK * D) // D
    kidx = 4 * jnp.arange(_PK)[:, None, None] + qidx[None, None, :]
    bsel = (jnp.arange(K)[None, :, None] == kidx).astype(_f32)
    item = _k4(ev0, agg0, wt0, x1p, a4, bsel, s4, bw0, b0t4,
               W0, b0.reshape(1, D), W1, b1.reshape(1, D))
    return (u, item[:, None, :])
